# baseline (speedup 1.0000x reference)
import functools
import math

import jax
import jax.numpy as jnp
from jax import lax
from jax.experimental import pallas as pl
from jax.experimental.pallas import tpu as pltpu
from jax.experimental.pallas import tpu_sc as plsc

_EMBED_DIM = 64
_WIDE = 128
_SCALE = math.sqrt(_EMBED_DIM)

_NC = 2
_NS = 16
_NW = _NC * _NS
_LANES = 16


def _make_kernel(V_pad, R, S):
  i_per_w = R // _NW
  stage_cnt = (96, 96, 104)
  stage_off = (0, 96, 192)
  assert sum(stage_cnt) == V_pad // _NS
  quads = S // 4

  mesh = plsc.VectorSubcoreMesh(core_axis_name="c", subcore_axis_name="s",
                                num_cores=_NC, num_subcores=_NS)

  @functools.partial(
      pl.kernel,
      mesh=mesh,
      compiler_params=pltpu.CompilerParams(needs_layout_passes=False),
      out_type=jax.ShapeDtypeStruct((S * _EMBED_DIM, R), jnp.float32),
      scratch_types=[
          pltpu.VMEM_SHARED((V_pad, _WIDE), jnp.float32),
          [pltpu.VMEM((i_per_w,), jnp.int32)] * 4,
          [pltpu.VMEM((i_per_w, _WIDE), jnp.float32)] * 2,
          [pltpu.VMEM((_EMBED_DIM, i_per_w), jnp.float32)] * 2,
          [pltpu.SemaphoreType.DMA] * 4,
          [pltpu.SemaphoreType.DMA] * 2,
          [pltpu.SemaphoreType.DMA] * 2,
      ],
  )
  def k(lut_hbm, idxt_hbm, out_hbm, table_sh, idxs, wides, trans,
        sems_i, sems_g, sems_w):
    cid = lax.axis_index("c")
    sid = lax.axis_index("s")
    wid = sid * _NC + cid

    tile_row0 = sid * (V_pad // _NS)
    for t in range(3):
      cnt = stage_cnt[t]
      row0 = tile_row0 + stage_off[t]
      stage = wides[0].at[pl.ds(0, cnt)]
      pltpu.sync_copy(lut_hbm.at[pl.ds(row0, cnt)], stage)

      def scale_row(i, _):
        for j in range(_WIDE // _LANES):
          wides[0][i, pl.ds(j * _LANES, _LANES)] = (
              wides[0][i, pl.ds(j * _LANES, _LANES)] * _SCALE)
        return 0

      lax.fori_loop(0, cnt, scale_row, 0)
      pltpu.sync_copy(stage, table_sh.at[pl.ds(row0, cnt)])
    plsc.subcore_barrier()

    i0 = wid * i_per_w
    lane_iota = lax.iota(jnp.int32, _LANES)

    def idx_copy(j, b):
      return pltpu.make_async_copy(
          idxt_hbm.at[pl.ds(j * R + i0, i_per_w)], idxs[b], sems_i[b])

    def gather_copy(b, w):
      return pltpu.make_async_copy(
          table_sh.at[idxs[b]], wides[w], sems_g[w])

    row_vecs = tuple(lane_iota + (c * _LANES)
                     for c in range(i_per_w // _LANES))
    zeros16 = lane_iota * 0

    def transpose(w):
      def tp_col(dh, _):
        d = dh * 2
        cols = zeros16 + d
        cols1 = cols + 1
        for c in range(i_per_w // _LANES):
          trans[w][d, pl.ds(c * _LANES, _LANES)] = plsc.load_gather(
              wides[w], [row_vecs[c], cols])
          trans[w][d + 1, pl.ds(c * _LANES, _LANES)] = plsc.load_gather(
              wides[w], [row_vecs[c], cols1])
        return 0
      lax.fori_loop(0, _EMBED_DIM // 2, tp_col, 0)


    def out_copy(j, w):
      return pltpu.make_async_copy(
          trans[w],
          out_hbm.at[pl.ds(j * _EMBED_DIM, _EMBED_DIM), pl.ds(i0, i_per_w)],
          sems_w[w])

    for b in range(3):
      idx_copy(b, b).start()

    def body(g, _):
      for bb in range(4):
        j = g * 4 + bb
        w, wp = bb % 2, (bb + 1) % 2
        idx_copy(j, bb).wait()

        if bb >= 2:
          out_copy(j, w).wait()
        else:
          @pl.when(g > 0)
          def _():
            out_copy(j, w).wait()

        gather_copy(bb, w).start()

        if bb >= 1:
          gather_copy((bb + 3) % 4, wp).wait()
          transpose(wp)
          out_copy(j - 1, wp).start()
        else:
          @pl.when(g > 0)
          def _():
            gather_copy((bb + 3) % 4, wp).wait()
            transpose(wp)
            out_copy(j - 1, wp).start()

        if bb == 0:
          idx_copy(j + 3, 3).start()
        else:
          @pl.when(j + 3 < S)
          def _():
            idx_copy(j + 3, (bb + 3) % 4).start()
      return 0

    lax.fori_loop(0, quads, body, 0)

    gather_copy(3, 1).wait()
    transpose(1)
    out_copy(S - 1, 1).start()
    out_copy(S - 2, 0).wait()
    out_copy(S - 1, 1).wait()

  return k


def kernel(x, lut):
  V, D = lut.shape
  R, S = x.shape
  V_pad = -(-V // (_NS * 8)) * (_NS * 8)
  lut_pad = jnp.pad(lut, ((0, V_pad - V), (0, _WIDE - D)))
  idx_t = x.T.astype(jnp.int32).reshape(-1)
  out2 = _make_kernel(V_pad, R, S)(lut_pad, idx_t)
  return out2.reshape(S, _EMBED_DIM, R).transpose(2, 0, 1)


# speedup vs baseline: 2.0600x; 2.0600x over previous
import functools
import math

import jax
import jax.numpy as jnp
from jax import lax
from jax.experimental import pallas as pl
from jax.experimental.pallas import tpu as pltpu
from jax.experimental.pallas import tpu_sc as plsc

_EMBED_DIM = 64
_WIDE = 128
_SCALE = math.sqrt(_EMBED_DIM)

_NC = 2
_NS = 16
_NW = _NC * _NS
_LANES = 16


def _make_kernel(V_pad, R, S):
  i_per_w = R // _NW
  stage_cnt = (96, 96, 104)
  stage_off = (0, 96, 192)
  assert sum(stage_cnt) == V_pad // _NS
  quads = S // 4

  mesh = plsc.VectorSubcoreMesh(core_axis_name="c", subcore_axis_name="s",
                                num_cores=_NC, num_subcores=_NS)

  @functools.partial(
      pl.kernel,
      mesh=mesh,
      compiler_params=pltpu.CompilerParams(needs_layout_passes=False),
      out_type=jax.ShapeDtypeStruct((S * _EMBED_DIM, R), jnp.float32),
      scratch_types=[
          pltpu.VMEM_SHARED((V_pad, _WIDE), jnp.float32),
          [pltpu.VMEM((i_per_w,), jnp.int32)] * 4,
          [pltpu.VMEM((i_per_w, _WIDE), jnp.float32)] * 2,
          [pltpu.VMEM((_EMBED_DIM, i_per_w), jnp.float32)] * 2,
          [pltpu.SemaphoreType.DMA] * 4,
          [pltpu.SemaphoreType.DMA] * 2,
          [pltpu.SemaphoreType.DMA] * 2,
      ],
  )
  def k(lut_hbm, idxt_hbm, out_hbm, table_sh, idxs, wides, trans,
        sems_i, sems_g, sems_w):
    cid = lax.axis_index("c")
    sid = lax.axis_index("s")
    wid = sid * _NC + cid

    tile_row0 = sid * (V_pad // _NS)
    for t in range(3):
      cnt = stage_cnt[t]
      row0 = tile_row0 + stage_off[t]
      stage = wides[0].at[pl.ds(0, cnt)]
      pltpu.sync_copy(lut_hbm.at[pl.ds(row0, cnt)], stage)

      def scale_row(i, _):
        for j in range(_WIDE // _LANES):
          wides[0][i, pl.ds(j * _LANES, _LANES)] = (
              wides[0][i, pl.ds(j * _LANES, _LANES)] * _SCALE)
        return 0

      lax.fori_loop(0, cnt, scale_row, 0)
      pltpu.sync_copy(stage, table_sh.at[pl.ds(row0, cnt)])
    plsc.subcore_barrier()

    i0 = wid * i_per_w
    lane_iota = lax.iota(jnp.int32, _LANES)

    def idx_copy(j, b):
      return pltpu.make_async_copy(
          idxt_hbm.at[pl.ds(j * R + i0, i_per_w)], idxs[b], sems_i[b])

    def gather_copy(b, w):
      return pltpu.make_async_copy(
          table_sh.at[idxs[b]], wides[w], sems_g[w])

    row_vecs = tuple(lane_iota + (c * _LANES)
                     for c in range(i_per_w // _LANES))
    perms = tuple((lane_iota + o) % _LANES for o in range(_LANES))

    def transpose(w):
      def tp_band(db, _):
        d0 = db * _LANES
        for o in range(_LANES):
          diag = perms[o] + d0
          for c in range(i_per_w // _LANES):
            v = plsc.load_gather(wides[w], [row_vecs[c], diag])
            plsc.store_scatter(trans[w], [diag, row_vecs[c]], v)
        return 0
      lax.fori_loop(0, _EMBED_DIM // _LANES, tp_band, 0)


    def out_copy(j, w):
      return pltpu.make_async_copy(
          trans[w],
          out_hbm.at[pl.ds(j * _EMBED_DIM, _EMBED_DIM), pl.ds(i0, i_per_w)],
          sems_w[w])

    for b in range(3):
      idx_copy(b, b).start()

    def body(g, _):
      for bb in range(4):
        j = g * 4 + bb
        w, wp = bb % 2, (bb + 1) % 2
        idx_copy(j, bb).wait()

        if bb >= 2:
          out_copy(j, w).wait()
        else:
          @pl.when(g > 0)
          def _():
            out_copy(j, w).wait()

        gather_copy(bb, w).start()

        if bb >= 1:
          gather_copy((bb + 3) % 4, wp).wait()
          transpose(wp)
          out_copy(j - 1, wp).start()
        else:
          @pl.when(g > 0)
          def _():
            gather_copy((bb + 3) % 4, wp).wait()
            transpose(wp)
            out_copy(j - 1, wp).start()

        if bb == 0:
          idx_copy(j + 3, 3).start()
        else:
          @pl.when(j + 3 < S)
          def _():
            idx_copy(j + 3, (bb + 3) % 4).start()
      return 0

    lax.fori_loop(0, quads, body, 0)

    gather_copy(3, 1).wait()
    transpose(1)
    out_copy(S - 1, 1).start()
    out_copy(S - 2, 0).wait()
    out_copy(S - 1, 1).wait()

  return k


def kernel(x, lut):
  V, D = lut.shape
  R, S = x.shape
  V_pad = -(-V // (_NS * 8)) * (_NS * 8)
  lut_pad = jnp.pad(lut, ((0, V_pad - V), (0, _WIDE - D)))
  idx_t = x.T.astype(jnp.int32).reshape(-1)
  out2 = _make_kernel(V_pad, R, S)(lut_pad, idx_t)
  return out2.reshape(S, _EMBED_DIM, R).transpose(2, 0, 1)


# speedup vs baseline: 2.4407x; 1.1848x over previous
import functools
import math

import jax
import jax.numpy as jnp
from jax import lax
from jax.experimental import pallas as pl
from jax.experimental.pallas import tpu as pltpu
from jax.experimental.pallas import tpu_sc as plsc

_EMBED_DIM = 64
_WIDE = 128
_SCALE = math.sqrt(_EMBED_DIM)

_NC = 2
_NS = 16
_NW = _NC * _NS

_H_OFF = (0, 96)
_H_CNT = (96, 104)
_BUF = 104


def _make_kernel(V_pad, R, S):
  r_per_w = R // _NW
  row_pairs = r_per_w // 2
  stage_cnt = (96, 96, 104)
  stage_off = (0, 96, 192)
  assert sum(stage_cnt) == V_pad // _NS

  mesh = plsc.VectorSubcoreMesh(core_axis_name="c", subcore_axis_name="s",
                                num_cores=_NC, num_subcores=_NS)

  @functools.partial(
      pl.kernel,
      mesh=mesh,
      out_type=jax.ShapeDtypeStruct((R, S, _EMBED_DIM), jnp.float32),
      scratch_types=[
          pltpu.VMEM_SHARED((V_pad, _WIDE), jnp.float32),
          [pltpu.VMEM((_BUF,), jnp.int32)] * 4,
          [pltpu.VMEM((_BUF, _WIDE), jnp.float32)] * 2,
          [pltpu.VMEM((_BUF, _EMBED_DIM), jnp.float32)] * 2,
          [pltpu.SemaphoreType.DMA] * 4,
          [pltpu.SemaphoreType.DMA] * 2,
          [pltpu.SemaphoreType.DMA] * 2,
      ],
  )
  def k(lut_hbm, idx_hbm, out_hbm, table_sh, idxs, wides, packs,
        sems_i, sems_g, sems_w):
    cid = lax.axis_index("c")
    sid = lax.axis_index("s")
    wid = sid * _NC + cid

    tile_row0 = sid * (V_pad // _NS)
    for t in range(3):
      cnt = stage_cnt[t]
      row0 = tile_row0 + stage_off[t]
      stage = wides[0].at[pl.ds(0, cnt)]
      pltpu.sync_copy(lut_hbm.at[pl.ds(row0, cnt)], stage)

      def scale_row(i, _):
        for j in range(_WIDE // 16):
          wides[0][i, pl.ds(j * 16, 16)] = (
              wides[0][i, pl.ds(j * 16, 16)] * _SCALE)
        return 0

      lax.fori_loop(0, cnt, scale_row, 0)
      pltpu.sync_copy(stage, table_sh.at[pl.ds(row0, cnt)])
    plsc.subcore_barrier()

    row_base = wid * r_per_w
    idx_base = wid * r_per_w * S

    def idx_copy(r, h, b):
      return pltpu.make_async_copy(
          idx_hbm.at[pl.ds(idx_base + r * S + _H_OFF[h], _H_CNT[h])],
          idxs[b].at[pl.ds(0, _H_CNT[h])], sems_i[b])

    def gather_copy(b, h):
      return pltpu.make_async_copy(
          table_sh.at[idxs[b].at[pl.ds(0, _H_CNT[h])]],
          wides[h].at[pl.ds(0, _H_CNT[h])], sems_g[h])

    def repack(h):
      def rp_row(i, _):
        for j in range(_EMBED_DIM // 16):
          packs[h][i, pl.ds(j * 16, 16)] = wides[h][i, pl.ds(j * 16, 16)]
        return 0
      lax.fori_loop(0, _H_CNT[h], rp_row, 0)

    def out_copy(r, h):
      return pltpu.make_async_copy(
          packs[h].at[pl.ds(0, _H_CNT[h])],
          out_hbm.at[row_base + r, pl.ds(_H_OFF[h], _H_CNT[h])],
          sems_w[h])

    for b in range(3):
      idx_copy(b // 2, b % 2, b).start()

    pf = ((1, 1, 3), (2, 0, 0), (2, 1, 1), (3, 0, 2))

    def body(g, _):
      for rr in range(2):
        r = g * 2 + rr
        for h in range(2):
          b = 2 * rr + h
          hp = 1 - h
          idx_copy(r, h, b).wait()

          if b >= 2:
            out_copy(r, h).wait()
          else:
            @pl.when(g > 0)
            def _():
              out_copy(r, h).wait()

          gather_copy(b, h).start()

          rp = r if h == 1 else r - 1
          if b >= 1:
            gather_copy((b + 3) % 4, hp).wait()
            repack(hp)
            out_copy(rp, hp).start()
          else:
            @pl.when(g > 0)
            def _():
              gather_copy((b + 3) % 4, hp).wait()
              repack(hp)
              out_copy(rp, hp).start()

          dr, nh, slot = pf[b]
          if b == 0:
            idx_copy(g * 2 + dr, nh, slot).start()
          else:
            @pl.when(g + 1 < row_pairs)
            def _():
              idx_copy(g * 2 + dr, nh, slot).start()
      return 0

    lax.fori_loop(0, row_pairs, body, 0)

    gather_copy(3, 1).wait()
    repack(1)
    out_copy(r_per_w - 1, 1).start()
    out_copy(r_per_w - 1, 0).wait()
    out_copy(r_per_w - 1, 1).wait()

  return k


def kernel(x, lut):
  V, D = lut.shape
  R, S = x.shape
  V_pad = -(-V // (_NS * 8)) * (_NS * 8)
  lut_pad = jnp.pad(lut, ((0, V_pad - V), (0, _WIDE - D)))
  idx_flat = x.reshape(-1).astype(jnp.int32)
  return _make_kernel(V_pad, R, S)(lut_pad, idx_flat)
